# Initial kernel scaffold; baseline (speedup 1.0000x reference)
import jax
import jax.numpy as jnp
from jax.experimental import pallas as pl


def kernel(x, edge_index, W1, b1, W2, b2):
    raise NotImplementedError("write your pallas kernel here")


# speedup vs baseline: 4.4485x; 4.4485x over previous
import jax
import jax.numpy as jnp
from jax import lax
from jax.experimental import pallas as pl
from jax.experimental.pallas import tpu as pltpu
from jax.experimental.pallas import tpu_sc as plsc

N = 10000
E = 320000
D = 128
K = 10
ALPHA = 0.1

L = 16
NS = 16
NP = 10240
CHUNK = 128
G = 8
CT = 160
EPT = CHUNK * CT
EP = EPT * NS
RT = NP // NS
CB = 64
RC = RT // CB


def _mlp_block(x_ref, w1_ref, b1_ref, w2_ref, b2_ref, o_ref):
    x = x_ref[...]
    h = lax.dot_general(x, w1_ref[...], (((1,), (1,)), ((), ())),
                        preferred_element_type=jnp.float32)
    h = jnp.maximum(h + b1_ref[...], 0.0)
    o = lax.dot_general(h, w2_ref[...], (((1,), (1,)), ((), ())),
                        preferred_element_type=jnp.float32)
    o_ref[...] = o + b2_ref[...]


def _mlp(xp, W1, b1, W2, b2):
    BR = 512
    return pl.pallas_call(
        _mlp_block,
        grid=(NP // BR,),
        in_specs=[
            pl.BlockSpec((BR, D), lambda i: (i, 0)),
            pl.BlockSpec((D, D), lambda i: (0, 0)),
            pl.BlockSpec((1, D), lambda i: (0, 0)),
            pl.BlockSpec((D, D), lambda i: (0, 0)),
            pl.BlockSpec((1, D), lambda i: (0, 0)),
        ],
        out_specs=pl.BlockSpec((BR, D), lambda i: (i, 0)),
        out_shape=jax.ShapeDtypeStruct((NP, D), jnp.float32),
    )(xp, W1, b1.reshape(1, D), W2, b2.reshape(1, D))


def _prop_body(rows_hbm, cols_hbm, h_hbm, x_out, y_hbm,
               acc_sp, deg_sp,
               ridx, cidx, gb, ca, ch, dinv_t, ones_t, zb):
    cid = lax.axis_index("c")
    sid = lax.axis_index("s")

    @pl.when(cid == 0)
    def _():
        base_c = sid * CT
        base_r = sid * RT

        zeros16 = jnp.zeros((L,), jnp.float32)
        ones16 = jnp.ones((L,), jnp.float32)

        def _z(i, c):
            zb[pl.ds(i * L, L)] = zeros16
            return c
        lax.fori_loop(0, RT // L, _z, 0)

        def _o(i, c):
            ones_t[pl.ds(i * L, L)] = ones16
            return c
        lax.fori_loop(0, CHUNK // L, _o, 0)

        pltpu.sync_copy(zb, deg_sp.at[pl.ds(base_r, RT)])
        plsc.subcore_barrier()

        def _degg(g, carry):
            pltpu.sync_copy(cols_hbm.at[pl.ds(base_c + g * G, G), :], cidx)

            def _deg(j, c2):
                pltpu.sync_copy(ones_t, deg_sp.at[cidx.at[j]], add=True)
                return c2
            lax.fori_loop(0, G, _deg, 0)
            return carry
        lax.fori_loop(0, CT // G, _degg, 0)
        plsc.subcore_barrier()

        pltpu.sync_copy(deg_sp.at[pl.ds(base_r, RT)], zb)

        def _rs(i, carry):
            d = zb[pl.ds(i * L, L)] + 1.0
            s = 0.5 * (d + 1.0)
            for _ in range(15):
                s = 0.5 * (s + d / s)
            dinv_t[pl.ds(i * L, L)] = 1.0 / s
            return carry
        lax.fori_loop(0, RT // L, _rs, 0)

        def _init(cj, carry):
            r0 = base_r + cj * CB
            pltpu.sync_copy(h_hbm.at[pl.ds(r0, CB), :], ch)

            def _row(rg, c2):
                dv = dinv_t[pl.ds(cj * CB + rg * L, L)]
                for j in range(L):
                    s = dv[j]
                    r = rg * L + j
                    for cc in range(D // L):
                        v = ch[r, pl.ds(cc * L, L)]
                        ch[r, pl.ds(cc * L, L)] = s * v
                return c2
            lax.fori_loop(0, CB // L, _row, 0)

            pltpu.sync_copy(ch, y_hbm.at[pl.ds(r0, CB), :])
            pltpu.sync_copy(ch, acc_sp.at[pl.ds(r0, CB), :])
            return carry
        lax.fori_loop(0, RC, _init, 0)

        def _step(k, carry):
            plsc.subcore_barrier()

            def _edgeg(g, c1):
                pltpu.sync_copy(rows_hbm.at[pl.ds(base_c + g * G, G), :], ridx)
                pltpu.sync_copy(cols_hbm.at[pl.ds(base_c + g * G, G), :], cidx)

                def _edge(j, c2):
                    pltpu.sync_copy(y_hbm.at[ridx.at[j]], gb)
                    pltpu.sync_copy(gb, acc_sp.at[cidx.at[j]], add=True)
                    return c2
                lax.fori_loop(0, G, _edge, 0)
                return c1
            lax.fori_loop(0, CT // G, _edgeg, 0)
            plsc.subcore_barrier()

            def _comb(cj, c3):
                r0 = base_r + cj * CB
                pltpu.sync_copy(acc_sp.at[pl.ds(r0, CB), :], ca)
                pltpu.sync_copy(h_hbm.at[pl.ds(r0, CB), :], ch)

                def _row(rg, c4):
                    dv = dinv_t[pl.ds(cj * CB + rg * L, L)]
                    for j in range(L):
                        s = dv[j]
                        a = (1.0 - ALPHA) * s
                        r = rg * L + j
                        for cc in range(D // L):
                            va = ca[r, pl.ds(cc * L, L)]
                            vh = ch[r, pl.ds(cc * L, L)]
                            xv = a * va + ALPHA * vh
                            ca[r, pl.ds(cc * L, L)] = xv
                            ch[r, pl.ds(cc * L, L)] = s * xv
                    return c4
                lax.fori_loop(0, CB // L, _row, 0)

                pltpu.sync_copy(ca, x_out.at[pl.ds(r0, CB), :])
                pltpu.sync_copy(ch, y_hbm.at[pl.ds(r0, CB), :])
                pltpu.sync_copy(ch, acc_sp.at[pl.ds(r0, CB), :])
                return c3
            lax.fori_loop(0, RC, _comb, 0)
            return carry
        lax.fori_loop(0, K, _step, 0)


def _propagate(rows_p, cols_p, h):
    mesh = plsc.VectorSubcoreMesh(core_axis_name="c", subcore_axis_name="s",
                                  num_cores=2, num_subcores=NS)
    out_type = (jax.ShapeDtypeStruct((NP, D), jnp.float32),
                jax.ShapeDtypeStruct((NP, D), jnp.float32))
    fn = pl.kernel(
        _prop_body,
        out_type,
        mesh=mesh,
        scratch_types=[
            pltpu.VMEM_SHARED((NP, D), jnp.float32),
            pltpu.VMEM_SHARED((NP,), jnp.float32),
            pltpu.VMEM((G, CHUNK), jnp.int32),
            pltpu.VMEM((G, CHUNK), jnp.int32),
            pltpu.VMEM((CHUNK, D), jnp.float32),
            pltpu.VMEM((CB, D), jnp.float32),
            pltpu.VMEM((CB, D), jnp.float32),
            pltpu.VMEM((RT,), jnp.float32),
            pltpu.VMEM((CHUNK,), jnp.float32),
            pltpu.VMEM((RT,), jnp.float32),
        ],
    )
    return fn(rows_p, cols_p, h)


def kernel(x, edge_index, W1, b1, W2, b2):
    xp = jnp.pad(x, ((0, NP - N), (0, 0)))
    h = _mlp(xp, W1, b1, W2, b2)

    rows = edge_index[0]
    cols = edge_index[1]
    pad = EP - E
    rows_p = jnp.concatenate(
        [rows, jnp.zeros((pad,), jnp.int32)]).reshape(EP // CHUNK, CHUNK)
    cols_p = jnp.concatenate(
        [cols, jnp.full((pad,), NP - 1, jnp.int32)]).reshape(EP // CHUNK, CHUNK)

    xk, _ = _propagate(rows_p, cols_p, h)
    return xk[:N]


# speedup vs baseline: 5.7819x; 1.2997x over previous
import jax
import jax.numpy as jnp
from jax import lax
from jax.experimental import pallas as pl
from jax.experimental.pallas import tpu as pltpu
from jax.experimental.pallas import tpu_sc as plsc

N = 10000
E = 320000
D = 128
K = 10
ALPHA = 0.1

L = 16
NS = 16
NP = 10240
CHUNK = 128
G = 16
CT = 160
EPT = CHUNK * CT
EP = EPT * NS
RT = NP // NS
CB = 128
RC = RT // CB


def _mlp_block(x_ref, w1_ref, b1_ref, w2_ref, b2_ref, o_ref):
    x = x_ref[...]
    h = lax.dot_general(x, w1_ref[...], (((1,), (1,)), ((), ())),
                        preferred_element_type=jnp.float32)
    h = jnp.maximum(h + b1_ref[...], 0.0)
    o = lax.dot_general(h, w2_ref[...], (((1,), (1,)), ((), ())),
                        preferred_element_type=jnp.float32)
    o_ref[...] = o + b2_ref[...]


def _mlp(xp, W1, b1, W2, b2):
    BR = 512
    return pl.pallas_call(
        _mlp_block,
        grid=(NP // BR,),
        in_specs=[
            pl.BlockSpec((BR, D), lambda i: (i, 0)),
            pl.BlockSpec((D, D), lambda i: (0, 0)),
            pl.BlockSpec((1, D), lambda i: (0, 0)),
            pl.BlockSpec((D, D), lambda i: (0, 0)),
            pl.BlockSpec((1, D), lambda i: (0, 0)),
        ],
        out_specs=pl.BlockSpec((BR, D), lambda i: (i, 0)),
        out_shape=jax.ShapeDtypeStruct((NP, D), jnp.float32),
    )(xp, W1, b1.reshape(1, D), W2, b2.reshape(1, D))


def _prop_body(rows_hbm, cols_hbm, h_hbm, x_out, y_hbm,
               acc_sp, deg_sp,
               ridx, cidx, ca, ch, dinv_t, ones_t, zb, gsem0, gsem1):
    cid = lax.axis_index("c")
    sid = lax.axis_index("s")

    @pl.when(cid == 0)
    def _():
        base_c = sid * CT
        base_r = sid * RT

        zeros16 = jnp.zeros((L,), jnp.float32)
        ones16 = jnp.ones((L,), jnp.float32)

        def _z(i, c):
            zb[pl.ds(i * L, L)] = zeros16
            return c
        lax.fori_loop(0, RT // L, _z, 0)

        def _o(i, c):
            ones_t[pl.ds(i * L, L)] = ones16
            return c
        lax.fori_loop(0, CHUNK // L, _o, 0)

        pltpu.sync_copy(zb, deg_sp.at[pl.ds(base_r, RT)])
        plsc.subcore_barrier()

        def _degg(g, carry):
            pltpu.sync_copy(cols_hbm.at[pl.ds(base_c + g * G, G), :], cidx)

            def _deg(j, c2):
                pltpu.sync_copy(ones_t, deg_sp.at[cidx.at[j]], add=True)
                return c2
            lax.fori_loop(0, G, _deg, 0)
            return carry
        lax.fori_loop(0, CT // G, _degg, 0)
        plsc.subcore_barrier()

        pltpu.sync_copy(deg_sp.at[pl.ds(base_r, RT)], zb)

        def _rs(i, carry):
            d = zb[pl.ds(i * L, L)] + 1.0
            s = 0.5 * (d + 1.0)
            for _ in range(15):
                s = 0.5 * (s + d / s)
            dinv_t[pl.ds(i * L, L)] = 1.0 / s
            return carry
        lax.fori_loop(0, RT // L, _rs, 0)

        def _init(cj, carry):
            r0 = base_r + cj * CB
            pltpu.sync_copy(h_hbm.at[pl.ds(r0, CB), :], ch)

            def _row(rg, c2):
                dv = dinv_t[pl.ds(cj * CB + rg * L, L)]
                for j in range(L):
                    s = dv[j]
                    r = rg * L + j
                    for cc in range(D // L):
                        v = ch[r, pl.ds(cc * L, L)]
                        ch[r, pl.ds(cc * L, L)] = s * v
                return c2
            lax.fori_loop(0, CB // L, _row, 0)

            pltpu.sync_copy(ch, y_hbm.at[pl.ds(r0, CB), :])
            pltpu.sync_copy(ch, acc_sp.at[pl.ds(r0, CB), :])
            return carry
        lax.fori_loop(0, RC, _init, 0)

        def _step(k, carry):
            plsc.subcore_barrier()

            bufs = (ca, ch)
            sems = (gsem0, gsem1)

            def _edgeg(g, c1):
                pltpu.sync_copy(rows_hbm.at[pl.ds(base_c + g * G, G), :], ridx)
                pltpu.sync_copy(cols_hbm.at[pl.ds(base_c + g * G, G), :], cidx)
                pltpu.async_copy(y_hbm.at[ridx.at[0]], bufs[0], sems[0])
                pltpu.async_copy(y_hbm.at[ridx.at[1]], bufs[1], sems[1])
                for c in range(G):
                    b = c % 2
                    pltpu.make_async_copy(y_hbm.at[ridx.at[c]], bufs[b],
                                          sems[b]).wait()
                    pltpu.sync_copy(bufs[b], acc_sp.at[cidx.at[c]], add=True)
                    if c + 2 < G:
                        pltpu.async_copy(y_hbm.at[ridx.at[c + 2]], bufs[b],
                                         sems[b])
                return c1
            lax.fori_loop(0, CT // G, _edgeg, 0)
            plsc.subcore_barrier()

            def _comb(cj, c3):
                r0 = base_r + cj * CB
                pltpu.sync_copy(acc_sp.at[pl.ds(r0, CB), :], ca)
                pltpu.sync_copy(h_hbm.at[pl.ds(r0, CB), :], ch)

                def _row(rg, c4):
                    dv = dinv_t[pl.ds(cj * CB + rg * L, L)]
                    for j in range(L):
                        s = dv[j]
                        a = (1.0 - ALPHA) * s
                        r = rg * L + j
                        for cc in range(D // L):
                            va = ca[r, pl.ds(cc * L, L)]
                            vh = ch[r, pl.ds(cc * L, L)]
                            xv = a * va + ALPHA * vh
                            ca[r, pl.ds(cc * L, L)] = xv
                            ch[r, pl.ds(cc * L, L)] = s * xv
                    return c4
                lax.fori_loop(0, CB // L, _row, 0)

                pltpu.sync_copy(ca, x_out.at[pl.ds(r0, CB), :])
                pltpu.sync_copy(ch, y_hbm.at[pl.ds(r0, CB), :])
                pltpu.sync_copy(ch, acc_sp.at[pl.ds(r0, CB), :])
                return c3
            lax.fori_loop(0, RC, _comb, 0)
            return carry
        lax.fori_loop(0, K, _step, 0)


def _propagate(rows_p, cols_p, h):
    mesh = plsc.VectorSubcoreMesh(core_axis_name="c", subcore_axis_name="s",
                                  num_cores=2, num_subcores=NS)
    out_type = (jax.ShapeDtypeStruct((NP, D), jnp.float32),
                jax.ShapeDtypeStruct((NP, D), jnp.float32))
    fn = pl.kernel(
        _prop_body,
        out_type,
        mesh=mesh,
        scratch_types=[
            pltpu.VMEM_SHARED((NP, D), jnp.float32),
            pltpu.VMEM_SHARED((NP,), jnp.float32),
            pltpu.VMEM((G, CHUNK), jnp.int32),
            pltpu.VMEM((G, CHUNK), jnp.int32),
            pltpu.VMEM((CB, D), jnp.float32),
            pltpu.VMEM((CB, D), jnp.float32),
            pltpu.VMEM((RT,), jnp.float32),
            pltpu.VMEM((CHUNK,), jnp.float32),
            pltpu.VMEM((RT,), jnp.float32),
            pltpu.SemaphoreType.DMA,
            pltpu.SemaphoreType.DMA,
        ],
    )
    return fn(rows_p, cols_p, h)


def kernel(x, edge_index, W1, b1, W2, b2):
    xp = jnp.pad(x, ((0, NP - N), (0, 0)))
    h = _mlp(xp, W1, b1, W2, b2)

    rows = edge_index[0]
    cols = edge_index[1]
    pad = EP - E
    rows_p = jnp.concatenate(
        [rows, jnp.zeros((pad,), jnp.int32)]).reshape(EP // CHUNK, CHUNK)
    cols_p = jnp.concatenate(
        [cols, jnp.full((pad,), NP - 1, jnp.int32)]).reshape(EP // CHUNK, CHUNK)

    xk, _ = _propagate(rows_p, cols_p, h)
    return xk[:N]
